# Initial kernel scaffold; baseline (speedup 1.0000x reference)
import jax
import jax.numpy as jnp
from jax.experimental import pallas as pl


def kernel(objectness, box_regression, anchors):
    raise NotImplementedError("write your pallas kernel here")


# speedup vs baseline: 4.4757x; 4.4757x over previous
import functools

import jax
import jax.numpy as jnp
import numpy as np
from jax.experimental import pallas as pl
from jax.experimental.pallas import tpu as pltpu

PRE_NMS_TOP_N = 1000
POST_NMS_TOP_N = 300
NMS_THRESH = 0.7
IMAGE_W, IMAGE_H = 800, 800
BBOX_XFORM_CLIP = float(np.log(1000.0 / 16.0))

M = 1024
GB = 2048
NUM = 120000
NUMP = 59 * GB
OUTW = 304
NMSB = 128


def _rpn_kernel(scores_ref, idx_ref, data_ref, out_ref, iou_scr):
    scores = scores_ref[0]
    idx = idx_ref[0]

    def gth(c, acc):
        d = data_ref[0, c]
        lane = jax.lax.broadcasted_iota(jnp.int32, (GB, M), 0) + c * GB
        oh = (lane == idx).astype(jnp.float32)
        return acc + jnp.dot(d, oh, preferred_element_type=jnp.float32)

    g = jax.lax.fori_loop(0, NUMP // GB, gth, jnp.zeros((8, M), jnp.float32))

    dx, dy = g[0:1], g[1:2]
    dw = jnp.minimum(g[2:3], BBOX_XFORM_CLIP)
    dh = jnp.minimum(g[3:4], BBOX_XFORM_CLIP)
    ax1, ay1, ax2, ay2 = g[4:5], g[5:6], g[6:7], g[7:8]

    widths = ax2 - ax1 + 1.0
    heights = ay2 - ay1 + 1.0
    ctr_x = ax1 + 0.5 * widths
    ctr_y = ay1 + 0.5 * heights
    pred_ctr_x = dx * widths + ctr_x
    pred_ctr_y = dy * heights + ctr_y
    pred_w = jnp.exp(dw) * widths
    pred_h = jnp.exp(dh) * heights
    x1 = jnp.clip(pred_ctr_x - 0.5 * pred_w, 0.0, IMAGE_W - 1.0)
    y1 = jnp.clip(pred_ctr_y - 0.5 * pred_h, 0.0, IMAGE_H - 1.0)
    x2 = jnp.clip(pred_ctr_x + 0.5 * pred_w - 1.0, 0.0, IMAGE_W - 1.0)
    y2 = jnp.clip(pred_ctr_y + 0.5 * pred_h - 1.0, 0.0, IMAGE_H - 1.0)

    pos = jax.lax.broadcasted_iota(jnp.int32, (1, M), 1)
    valid = ((x2 - x1 + 1.0 >= 0.0) & (y2 - y1 + 1.0 >= 0.0)
             & (pos < PRE_NMS_TOP_N))
    area = (x2 - x1 + 1.0) * (y2 - y1 + 1.0)

    for t in range(M // NMSB):
        sl = slice(t * NMSB, (t + 1) * NMSB)
        x1c = x1[0, sl].reshape(NMSB, 1)
        y1c = y1[0, sl].reshape(NMSB, 1)
        x2c = x2[0, sl].reshape(NMSB, 1)
        y2c = y2[0, sl].reshape(NMSB, 1)
        ac = area[0, sl].reshape(NMSB, 1)
        w = jnp.maximum(jnp.minimum(x2c, x2) - jnp.maximum(x1c, x1) + 1.0, 0.0)
        h = jnp.maximum(jnp.minimum(y2c, y2) - jnp.maximum(y1c, y1) + 1.0, 0.0)
        inter = w * h
        iou = inter / (ac + area - inter)
        iou_scr[sl, :] = (iou > NMS_THRESH).astype(jnp.float32)

    keep = valid.astype(jnp.float32)

    for b in range(M // NMSB):
        bs = b * NMSB
        ksub = keep[0:1, bs:bs + NMSB]
        loc = jax.lax.broadcasted_iota(jnp.int32, (1, NMSB), 1)
        sub = iou_scr[bs:bs + NMSB, bs:bs + NMSB]
        rowsel = jax.lax.broadcasted_iota(jnp.int32, (NMSB, NMSB), 0)

        def inner(i, ks):
            row = jnp.sum(sub * (rowsel == i).astype(jnp.float32),
                          axis=0, keepdims=True)
            kg = jnp.sum(ks * (loc == i).astype(jnp.float32))
            gt = (loc > i).astype(jnp.float32)
            return ks * (1.0 - kg * row * gt)

        ksub = jax.lax.fori_loop(0, NMSB, inner, ksub)

        rows = iou_scr[bs:bs + NMSB, :]
        kcol = ksub.reshape(NMSB, 1)
        supc = jnp.max(rows * kcol, axis=0, keepdims=True)
        later = (pos >= bs + NMSB).astype(jnp.float32)
        keep = keep * (1.0 - supc * later)
        parts = []
        if bs > 0:
            parts.append(keep[0:1, :bs])
        parts.append(ksub)
        if bs + NMSB < M:
            parts.append(keep[0:1, bs + NMSB:])
        keep = jnp.concatenate(parts, axis=1)

    tri = (jax.lax.broadcasted_iota(jnp.int32, (M, M), 0)
           <= jax.lax.broadcasted_iota(jnp.int32, (M, M), 1)).astype(jnp.float32)
    csk = jnp.dot(keep, tri, preferred_element_type=jnp.float32)
    k_total = csk[0, M - 1]
    posf = pos.astype(jnp.float32)
    csnk = (posf + 1.0) - csk
    slot = keep * (csk - 1.0) + (1.0 - keep) * (k_total + csnk - 1.0)

    iota_r = jax.lax.broadcasted_iota(jnp.int32, (M, OUTW), 1).astype(jnp.float32)
    sel = (slot.reshape(M, 1) == iota_r).astype(jnp.float32)

    smask = keep * scores + (1.0 - keep) * (-1e10)
    payload = jnp.concatenate(
        [x1, y1, x2, y2, smask, jnp.zeros((3, M), jnp.float32)], axis=0)
    out_ref[0] = jnp.dot(payload, sel, preferred_element_type=jnp.float32)


@jax.jit
def kernel(objectness, box_regression, anchors):
    N, A, H, W = objectness.shape
    obj = objectness.transpose(0, 2, 3, 1).reshape(N, -1)
    scores_all = jax.nn.sigmoid(obj)
    scores, idx = jax.lax.top_k(scores_all, PRE_NMS_TOP_N)

    scores_p = jnp.pad(scores, ((0, 0), (0, M - PRE_NMS_TOP_N)),
                       constant_values=-1e10).reshape(N, 1, M)
    idx_p = jnp.pad(idx, ((0, 0), (0, M - PRE_NMS_TOP_N))).reshape(N, 1, M)

    br = box_regression.reshape(N, A, 4, H, W).transpose(0, 3, 4, 1, 2)
    br = br.reshape(N, -1, 4)
    data = jnp.concatenate([br, anchors], axis=-1)
    data = jnp.pad(data, ((0, 0), (0, NUMP - NUM), (0, 0)))
    data = data.reshape(N, NUMP // GB, GB, 8).transpose(0, 1, 3, 2)

    out = pl.pallas_call(
        _rpn_kernel,
        grid=(N,),
        in_specs=[
            pl.BlockSpec((1, 1, M), lambda n: (n, 0, 0)),
            pl.BlockSpec((1, 1, M), lambda n: (n, 0, 0)),
            pl.BlockSpec((1, NUMP // GB, 8, GB), lambda n: (n, 0, 0, 0)),
        ],
        out_specs=pl.BlockSpec((1, 8, OUTW), lambda n: (n, 0, 0)),
        out_shape=jax.ShapeDtypeStruct((N, 8, OUTW), jnp.float32),
        scratch_shapes=[pltpu.VMEM((M, M), jnp.float32)],
    )(scores_p, idx_p, data)

    boxes = out[:, :4, :POST_NMS_TOP_N].transpose(0, 2, 1)
    fs = out[:, 4, :POST_NMS_TOP_N]
    return jnp.concatenate([boxes, fs[:, :, None]], axis=-1)


# speedup vs baseline: 4.8332x; 1.0799x over previous
import functools

import jax
import jax.numpy as jnp
import numpy as np
from jax.experimental import pallas as pl
from jax.experimental.pallas import tpu as pltpu

PRE_NMS_TOP_N = 1000
POST_NMS_TOP_N = 300
NMS_THRESH = 0.7
IMAGE_W, IMAGE_H = 800, 800
BBOX_XFORM_CLIP = float(np.log(1000.0 / 16.0))

M = 1024
GB = 2048
NUM = 120000
NUMP = 59 * GB
OUTW = 304
NMSB = 128


def _rpn_kernel(scores_ref, idx_ref, data_ref, out_ref, iou_scr):
    scores = scores_ref[0]
    idx = idx_ref[0]

    def gth(c, acc):
        d = data_ref[0, c]
        lane = jax.lax.broadcasted_iota(jnp.int32, (GB, M), 0) + c * GB
        oh = (lane == idx).astype(jnp.float32)
        return acc + jnp.dot(d, oh, preferred_element_type=jnp.float32)

    g = jax.lax.fori_loop(0, NUMP // GB, gth, jnp.zeros((8, M), jnp.float32))

    dx, dy = g[0:1], g[1:2]
    dw = jnp.minimum(g[2:3], BBOX_XFORM_CLIP)
    dh = jnp.minimum(g[3:4], BBOX_XFORM_CLIP)
    ax1, ay1, ax2, ay2 = g[4:5], g[5:6], g[6:7], g[7:8]

    widths = ax2 - ax1 + 1.0
    heights = ay2 - ay1 + 1.0
    ctr_x = ax1 + 0.5 * widths
    ctr_y = ay1 + 0.5 * heights
    pred_ctr_x = dx * widths + ctr_x
    pred_ctr_y = dy * heights + ctr_y
    pred_w = jnp.exp(dw) * widths
    pred_h = jnp.exp(dh) * heights
    x1 = jnp.clip(pred_ctr_x - 0.5 * pred_w, 0.0, IMAGE_W - 1.0)
    y1 = jnp.clip(pred_ctr_y - 0.5 * pred_h, 0.0, IMAGE_H - 1.0)
    x2 = jnp.clip(pred_ctr_x + 0.5 * pred_w - 1.0, 0.0, IMAGE_W - 1.0)
    y2 = jnp.clip(pred_ctr_y + 0.5 * pred_h - 1.0, 0.0, IMAGE_H - 1.0)

    pos = jax.lax.broadcasted_iota(jnp.int32, (1, M), 1)
    valid = ((x2 - x1 + 1.0 >= 0.0) & (y2 - y1 + 1.0 >= 0.0)
             & (pos < PRE_NMS_TOP_N))
    area = (x2 - x1 + 1.0) * (y2 - y1 + 1.0)

    for t in range(M // NMSB):
        sl = slice(t * NMSB, (t + 1) * NMSB)
        x1c = x1[0, sl].reshape(NMSB, 1)
        y1c = y1[0, sl].reshape(NMSB, 1)
        x2c = x2[0, sl].reshape(NMSB, 1)
        y2c = y2[0, sl].reshape(NMSB, 1)
        ac = area[0, sl].reshape(NMSB, 1)
        w = jnp.maximum(jnp.minimum(x2c, x2) - jnp.maximum(x1c, x1) + 1.0, 0.0)
        h = jnp.maximum(jnp.minimum(y2c, y2) - jnp.maximum(y1c, y1) + 1.0, 0.0)
        inter = w * h
        iou = inter / (ac + area - inter)
        iou_scr[sl, :] = (iou > NMS_THRESH).astype(jnp.float32)

    keep = valid.astype(jnp.float32)

    for b in range(M // NMSB):
        bs = b * NMSB
        ksub = keep[0:1, bs:bs + NMSB]
        loc = jax.lax.broadcasted_iota(jnp.int32, (1, NMSB), 1)
        sub = iou_scr[bs:bs + NMSB, bs:bs + NMSB]

        for i in range(NMSB - 1):
            row = sub[i:i + 1, :]
            kg = ksub[0:1, i:i + 1]
            gt = (loc > i).astype(jnp.float32)
            ksub = ksub * (1.0 - kg * row * gt)

        rows = iou_scr[bs:bs + NMSB, :]
        kcol = ksub.reshape(NMSB, 1)
        supc = jnp.max(rows * kcol, axis=0, keepdims=True)
        later = (pos >= bs + NMSB).astype(jnp.float32)
        keep = keep * (1.0 - supc * later)
        parts = []
        if bs > 0:
            parts.append(keep[0:1, :bs])
        parts.append(ksub)
        if bs + NMSB < M:
            parts.append(keep[0:1, bs + NMSB:])
        keep = jnp.concatenate(parts, axis=1)

    tri = (jax.lax.broadcasted_iota(jnp.int32, (M, M), 0)
           <= jax.lax.broadcasted_iota(jnp.int32, (M, M), 1)).astype(jnp.float32)
    csk = jnp.dot(keep, tri, preferred_element_type=jnp.float32)
    k_total = csk[0, M - 1]
    posf = pos.astype(jnp.float32)
    csnk = (posf + 1.0) - csk
    slot = keep * (csk - 1.0) + (1.0 - keep) * (k_total + csnk - 1.0)

    iota_r = jax.lax.broadcasted_iota(jnp.int32, (M, OUTW), 1).astype(jnp.float32)
    sel = (slot.reshape(M, 1) == iota_r).astype(jnp.float32)

    smask = keep * scores + (1.0 - keep) * (-1e10)
    payload = jnp.concatenate(
        [x1, y1, x2, y2, smask, jnp.zeros((3, M), jnp.float32)], axis=0)
    out_ref[0] = jnp.dot(payload, sel, preferred_element_type=jnp.float32)


@jax.jit
def kernel(objectness, box_regression, anchors):
    N, A, H, W = objectness.shape
    obj = objectness.transpose(0, 2, 3, 1).reshape(N, -1)
    scores_all = jax.nn.sigmoid(obj)
    scores, idx = jax.lax.top_k(scores_all, PRE_NMS_TOP_N)

    scores_p = jnp.pad(scores, ((0, 0), (0, M - PRE_NMS_TOP_N)),
                       constant_values=-1e10).reshape(N, 1, M)
    idx_p = jnp.pad(idx, ((0, 0), (0, M - PRE_NMS_TOP_N))).reshape(N, 1, M)

    br = box_regression.reshape(N, A, 4, H, W).transpose(0, 3, 4, 1, 2)
    br = br.reshape(N, -1, 4)
    data = jnp.concatenate([br, anchors], axis=-1)
    data = jnp.pad(data, ((0, 0), (0, NUMP - NUM), (0, 0)))
    data = data.reshape(N, NUMP // GB, GB, 8).transpose(0, 1, 3, 2)

    out = pl.pallas_call(
        _rpn_kernel,
        grid=(N,),
        in_specs=[
            pl.BlockSpec((1, 1, M), lambda n: (n, 0, 0)),
            pl.BlockSpec((1, 1, M), lambda n: (n, 0, 0)),
            pl.BlockSpec((1, NUMP // GB, 8, GB), lambda n: (n, 0, 0, 0)),
        ],
        out_specs=pl.BlockSpec((1, 8, OUTW), lambda n: (n, 0, 0)),
        out_shape=jax.ShapeDtypeStruct((N, 8, OUTW), jnp.float32),
        scratch_shapes=[pltpu.VMEM((M, M), jnp.float32)],
    )(scores_p, idx_p, data)

    boxes = out[:, :4, :POST_NMS_TOP_N].transpose(0, 2, 1)
    fs = out[:, 4, :POST_NMS_TOP_N]
    return jnp.concatenate([boxes, fs[:, :, None]], axis=-1)
